# baseline (speedup 1.0000x reference)
import jax
import jax.numpy as jnp
from jax.experimental import pallas as pl
from jax.experimental.pallas import tpu as pltpu

Q = 4096
K = 16384
DA = 24
CW = 512
NC = K // CW
LANES = 128


def _tree_min(vals):
    while len(vals) > 1:
        pairs = [jnp.minimum(vals[i], vals[i + 1])
                 for i in range(0, len(vals) - 1, 2)]
        if len(vals) % 2:
            pairs.append(vals[-1])
        vals = pairs
    return vals[0]


def _loss_body(x_ref, y_ref, out_ref):
    x = x_ref[...]
    y = y_ref[...]
    x2 = jnp.sum(x * x, axis=1, keepdims=True)
    y2 = jnp.sum(y * y, axis=1, keepdims=True)
    xa = jnp.concatenate(
        [x * -2.0, x2, jnp.ones((Q, 1), jnp.float32),
         jnp.zeros((Q, DA - 18), jnp.float32)], axis=1)
    ya = jnp.concatenate(
        [y, jnp.ones((K, 1), jnp.float32), y2,
         jnp.zeros((K, DA - 18), jnp.float32)], axis=1)

    xaccs = []
    ysum = jnp.float32(0.0)
    for c in range(NC):
        mm = jax.lax.dot_general(
            xa, jax.lax.slice(ya, (c * CW, 0), ((c + 1) * CW, DA)),
            (((1,), (1,)), ((), ())),
            preferred_element_type=jnp.float32,
            precision=jax.lax.Precision.DEFAULT)
        slabs = [mm[:, s * LANES:(s + 1) * LANES] for s in range(CW // LANES)]
        xaccs.append(_tree_min(slabs))
        miny = jnp.min(mm, axis=0)
        ysum = ysum + jnp.sum(jnp.sqrt(jnp.maximum(miny, 0.0)))

    xacc = _tree_min(xaccs)
    rowmin = jnp.min(xacc, axis=1)
    d1 = jnp.mean(jnp.sqrt(jnp.maximum(rowmin, 0.0)))
    out_ref[0, 0] = d1 + ysum / K


def kernel(X, Y):
    out = pl.pallas_call(
        _loss_body,
        grid=(1,),
        in_specs=[
            pl.BlockSpec((Q, 16), lambda k: (0, 0)),
            pl.BlockSpec((K, 16), lambda k: (0, 0)),
        ],
        out_specs=pl.BlockSpec(memory_space=pltpu.SMEM),
        out_shape=jax.ShapeDtypeStruct((1, 1), jnp.float32),
    )(X, Y)
    return out[0, 0]


# speedup vs baseline: 1.0712x; 1.0712x over previous
import jax
import jax.numpy as jnp
from jax.experimental import pallas as pl
from jax.experimental.pallas import tpu as pltpu

Q = 4096
K = 16384
DA = 24
CW = 512
NC = K // CW
LANES = 128


def _tree_min(vals):
    while len(vals) > 1:
        pairs = [jnp.minimum(vals[i], vals[i + 1])
                 for i in range(0, len(vals) - 1, 2)]
        if len(vals) % 2:
            pairs.append(vals[-1])
        vals = pairs
    return vals[0]


def _loss_body(xa_ref, ya_ref, out_ref):
    xa = xa_ref[...]

    xaccs = []
    ysum = jnp.float32(0.0)
    for c in range(NC):
        mm = jax.lax.dot_general(
            xa, ya_ref[c * CW:(c + 1) * CW, :], (((1,), (1,)), ((), ())),
            preferred_element_type=jnp.float32,
            precision=jax.lax.Precision.DEFAULT)
        mb = mm.astype(jnp.bfloat16)
        slabs = [mb[:, s * LANES:(s + 1) * LANES] for s in range(CW // LANES)]
        xaccs.append(_tree_min(slabs))
        miny = jnp.min(mb, axis=0).astype(jnp.float32)
        ysum = ysum + jnp.sum(jnp.sqrt(jnp.maximum(miny, 0.0)))

    xacc = _tree_min(xaccs)
    rowmin = jnp.min(xacc, axis=1).astype(jnp.float32)
    d1 = jnp.mean(jnp.sqrt(jnp.maximum(rowmin, 0.0)))
    out_ref[0, 0] = d1 + ysum / K


def _split_bf16(v):
    hi = v.astype(jnp.bfloat16)
    lo = (v - hi.astype(jnp.float32)).astype(jnp.bfloat16)
    return hi, lo


def kernel(X, Y):
    x2h, x2l = _split_bf16(jnp.sum(X * X, axis=1, keepdims=True))
    y2h, y2l = _split_bf16(jnp.sum(Y * Y, axis=1, keepdims=True))
    ones_x = jnp.ones((Q, 1), jnp.bfloat16)
    ones_y = jnp.ones((K, 1), jnp.bfloat16)
    pad_x = jnp.zeros((Q, DA - 20), jnp.bfloat16)
    pad_y = jnp.zeros((K, DA - 20), jnp.bfloat16)
    Xa = jnp.concatenate(
        [(X * -2.0).astype(jnp.bfloat16), x2h, x2l, ones_x, ones_x, pad_x],
        axis=1)
    Ya = jnp.concatenate(
        [Y.astype(jnp.bfloat16), ones_y, ones_y, y2h, y2l, pad_y],
        axis=1)

    out = pl.pallas_call(
        _loss_body,
        grid=(1,),
        in_specs=[
            pl.BlockSpec((Q, DA), lambda k: (0, 0)),
            pl.BlockSpec((K, DA), lambda k: (0, 0)),
        ],
        out_specs=pl.BlockSpec(memory_space=pltpu.SMEM),
        out_shape=jax.ShapeDtypeStruct((1, 1), jnp.float32),
    )(Xa, Ya)
    return out[0, 0]


# speedup vs baseline: 1.1108x; 1.0370x over previous
import jax
import jax.numpy as jnp
from jax.experimental import pallas as pl
from jax.experimental.pallas import tpu as pltpu

Q = 4096
K = 16384
DA = 24
KT = 4096
NK = K // KT
CW = 512
NC = KT // CW
LANES = 128


def _loss_body(xa_ref, ya_ref, out_ref, minx_ref, accy_ref):
    k = pl.program_id(0)
    inf = jnp.float32(jnp.inf)
    xa = xa_ref[...]

    xacc = jnp.where(k == 0, inf, minx_ref[...])
    ysum = jnp.float32(0.0)
    for c in range(NC):
        mm = jax.lax.dot_general(
            xa, ya_ref[c * CW:(c + 1) * CW, :], (((1,), (1,)), ((), ())),
            preferred_element_type=jnp.float32,
            precision=jax.lax.Precision.DEFAULT)
        fold = mm[:, :LANES]
        for s in range(1, CW // LANES):
            fold = jnp.minimum(fold, mm[:, s * LANES:(s + 1) * LANES])
        xacc = jnp.minimum(xacc, fold)
        miny = jnp.min(mm, axis=0)
        ysum = ysum + jnp.sum(jnp.sqrt(jnp.maximum(miny, 0.0)))
    minx_ref[...] = xacc
    accy_ref[0, 0] = jnp.where(k == 0, 0.0, accy_ref[0, 0]) + ysum

    @pl.when(k == NK - 1)
    def _finish():
        rowmin = jnp.min(minx_ref[...], axis=1)
        d1 = jnp.mean(jnp.sqrt(jnp.maximum(rowmin, 0.0)))
        out_ref[0, 0] = d1 + accy_ref[0, 0] / K


def kernel(X, Y):
    x2 = jnp.sum(X * X, axis=1, keepdims=True)
    y2 = jnp.sum(Y * Y, axis=1, keepdims=True)
    ones_x = jnp.ones((Q, 1), jnp.float32)
    ones_y = jnp.ones((K, 1), jnp.float32)
    pad_x = jnp.zeros((Q, DA - 18), jnp.float32)
    pad_y = jnp.zeros((K, DA - 18), jnp.float32)
    Xa = jnp.concatenate([X * -2.0, x2, ones_x, pad_x], axis=1)
    Ya = jnp.concatenate([Y, ones_y, y2, pad_y], axis=1)

    out = pl.pallas_call(
        _loss_body,
        grid=(NK,),
        in_specs=[
            pl.BlockSpec((Q, DA), lambda k: (0, 0)),
            pl.BlockSpec((KT, DA), lambda k: (k, 0)),
        ],
        out_specs=pl.BlockSpec(memory_space=pltpu.SMEM),
        out_shape=jax.ShapeDtypeStruct((1, 1), jnp.float32),
        scratch_shapes=[
            pltpu.VMEM((Q, LANES), jnp.float32),
            pltpu.SMEM((1, 1), jnp.float32),
        ],
    )(Xa, Ya)
    return out[0, 0]


# speedup vs baseline: 1.1124x; 1.0015x over previous
import jax
import jax.numpy as jnp
from jax.experimental import pallas as pl
from jax.experimental.pallas import tpu as pltpu

Q = 4096
K = 16384
DA = 24
KT = 8192
NK = K // KT
CW = 512
NC = KT // CW
LANES = 128


def _loss_body(xa_ref, ya_ref, out_ref, minx_ref, accy_ref):
    k = pl.program_id(0)
    inf = jnp.float32(jnp.inf)
    xa = xa_ref[...]

    xacc = jnp.where(k == 0, inf, minx_ref[...])
    ysum = jnp.float32(0.0)
    for c in range(NC):
        mm = jax.lax.dot_general(
            xa, ya_ref[c * CW:(c + 1) * CW, :], (((1,), (1,)), ((), ())),
            preferred_element_type=jnp.float32,
            precision=jax.lax.Precision.DEFAULT)
        fold = mm[:, :LANES]
        for s in range(1, CW // LANES):
            fold = jnp.minimum(fold, mm[:, s * LANES:(s + 1) * LANES])
        xacc = jnp.minimum(xacc, fold)
        miny = jnp.min(mm, axis=0)
        ysum = ysum + jnp.sum(jnp.sqrt(jnp.maximum(miny, 0.0)))
    minx_ref[...] = xacc
    accy_ref[0, 0] = jnp.where(k == 0, 0.0, accy_ref[0, 0]) + ysum

    @pl.when(k == NK - 1)
    def _finish():
        rowmin = jnp.min(minx_ref[...], axis=1)
        d1 = jnp.mean(jnp.sqrt(jnp.maximum(rowmin, 0.0)))
        out_ref[0, 0] = d1 + accy_ref[0, 0] / K


def kernel(X, Y):
    x2 = jnp.sum(X * X, axis=1, keepdims=True)
    y2 = jnp.sum(Y * Y, axis=1, keepdims=True)
    ones_x = jnp.ones((Q, 1), jnp.float32)
    ones_y = jnp.ones((K, 1), jnp.float32)
    pad_x = jnp.zeros((Q, DA - 18), jnp.float32)
    pad_y = jnp.zeros((K, DA - 18), jnp.float32)
    Xa = jnp.concatenate([X * -2.0, x2, ones_x, pad_x], axis=1)
    Ya = jnp.concatenate([Y, ones_y, y2, pad_y], axis=1)

    out = pl.pallas_call(
        _loss_body,
        grid=(NK,),
        in_specs=[
            pl.BlockSpec((Q, DA), lambda k: (0, 0)),
            pl.BlockSpec((KT, DA), lambda k: (k, 0)),
        ],
        out_specs=pl.BlockSpec(memory_space=pltpu.SMEM),
        out_shape=jax.ShapeDtypeStruct((1, 1), jnp.float32),
        scratch_shapes=[
            pltpu.VMEM((Q, LANES), jnp.float32),
            pltpu.SMEM((1, 1), jnp.float32),
        ],
    )(Xa, Ya)
    return out[0, 0]
